# Initial kernel scaffold; baseline (speedup 1.0000x reference)
import jax
import jax.numpy as jnp
from jax.experimental import pallas as pl


def kernel(x, edge_index, edge_weight, W1, b1, W2, b2, W3, b3, W4, b4, W5, b5):
    raise NotImplementedError("write your pallas kernel here")


# speedup vs baseline: 4.6898x; 4.6898x over previous
import dataclasses
import functools

import jax
import jax.numpy as jnp
from jax import lax
from jax.experimental import pallas as pl
from jax.experimental.pallas import tpu as pltpu
from jax.experimental.pallas import tpu_sc as plsc

_N = 10000
_E = 320000
_NC = 2
_NS = 16
_NW = _NC * _NS
_L = 16
_EB = 128
_NBLK = _E // _EB
_BPW = -(-_NBLK // _NW)
_RPS = 624
_REM = _N - _NS * _RPS
_ZB = 104


def _sc_spmm(support, src, dst, ew):
    n, d = support.shape
    assert n == _N and d % _L == 0
    mesh = plsc.VectorSubcoreMesh(core_axis_name="c", subcore_axis_name="s")
    cp = pltpu.CompilerParams()
    if "needs_layout_passes" in pltpu.CompilerParams.__dataclass_fields__:
        cp = dataclasses.replace(cp, needs_layout_passes=False)
    if d < 128 and "use_tc_tiling_on_sc" in pltpu.CompilerParams.__dataclass_fields__:
        cp = dataclasses.replace(cp, use_tc_tiling_on_sc=False)

    @functools.partial(
        pl.kernel,
        mesh=mesh,
        compiler_params=cp,
        out_type=jax.ShapeDtypeStruct((_NC, _N, d), jnp.float32),
        scratch_types=[
            pltpu.VMEM((_EB,), jnp.int32),
            pltpu.VMEM((_EB,), jnp.int32),
            pltpu.VMEM((_EB,), jnp.float32),
            pltpu.VMEM((_EB, d), jnp.float32),
            pltpu.VMEM_SHARED((_N, d), jnp.float32),
        ],
    )
    def spmm_kernel(sup_hbm, src_hbm, dst_hbm, ew_hbm, out_hbm,
                    src_v, dst_v, ew_v, rows_v, acc_sh):
        c = lax.axis_index("c")
        s = lax.axis_index("s")
        wid = s * _NC + c

        @pl.loop(0, _EB)
        def _(r):
            @pl.loop(0, d, step=_L)
            def _(col):
                rows_v[r, pl.ds(col, _L)] = jnp.zeros((_L,), jnp.float32)

        @pl.loop(0, _RPS, step=_ZB)
        def _(j):
            pltpu.sync_copy(rows_v.at[pl.ds(0, _ZB)],
                            acc_sh.at[pl.ds(s * _RPS + j, _ZB)])

        @pl.when(s == _NS - 1)
        def _():
            pltpu.sync_copy(rows_v.at[pl.ds(0, _REM)],
                            acc_sh.at[pl.ds(_NS * _RPS, _REM)])

        plsc.subcore_barrier()

        @pl.loop(0, _BPW)
        def _(j):
            blk = wid + j * _NW

            @pl.when(blk < _NBLK)
            def _():
                base = blk * _EB
                pltpu.sync_copy(src_hbm.at[pl.ds(base, _EB)], src_v)
                pltpu.sync_copy(dst_hbm.at[pl.ds(base, _EB)], dst_v)
                pltpu.sync_copy(ew_hbm.at[pl.ds(base, _EB)], ew_v)
                pltpu.sync_copy(sup_hbm.at[src_v], rows_v)

                @pl.loop(0, _EB)
                def _(e):
                    w = plsc.load_gather(ew_v, [jnp.full((_L,), e, jnp.int32)])

                    @pl.loop(0, d, step=_L)
                    def _(col):
                        rows_v[e, pl.ds(col, _L)] = rows_v[e, pl.ds(col, _L)] * w

                pltpu.sync_copy(rows_v, acc_sh.at[dst_v], add=True)

        plsc.subcore_barrier()

        pltpu.sync_copy(acc_sh.at[pl.ds(s * _RPS, _RPS)],
                        out_hbm.at[c, pl.ds(s * _RPS, _RPS)])

        @pl.when(s == _NS - 1)
        def _():
            pltpu.sync_copy(acc_sh.at[pl.ds(_NS * _RPS, _REM)],
                            out_hbm.at[c, pl.ds(_NS * _RPS, _REM)])

    return spmm_kernel(support, src, dst, ew)


_RB = 2000


def _part_spec(d):
    return pl.BlockSpec((_NC, _RB, d), lambda i: (0, i, 0))


def _full_spec(shape):
    nd = len(shape)
    return pl.BlockSpec(shape, lambda i: (0,) * nd)


def _tc_stage1(px, w1, b1, w2):
    def body(p_ref, w1_ref, b1_ref, w2_ref, oa_ref, ob_ref):
        a = p_ref[0] + p_ref[1]
        h = jnp.dot(a, w1_ref[...], preferred_element_type=jnp.float32)
        h = jnp.maximum(h + b1_ref[...], 0.0)
        s2 = jnp.dot(h, w2_ref[...], preferred_element_type=jnp.float32)
        oa_ref[...] = s2[:, :128]
        ob_ref[...] = s2[:, 128:]

    return pl.pallas_call(
        body,
        grid=(_N // _RB,),
        in_specs=[_part_spec(128), _full_spec((128, 512)),
                  _full_spec((1, 512)), _full_spec((512, 256))],
        out_specs=[pl.BlockSpec((_RB, 128), lambda i: (i, 0)),
                   pl.BlockSpec((_RB, 128), lambda i: (i, 0))],
        out_shape=[jax.ShapeDtypeStruct((_N, 128), jnp.float32),
                   jax.ShapeDtypeStruct((_N, 128), jnp.float32)],
    )(px, w1, b1, w2)


def _tc_stage2(pa, pb, b2, w3a, w3b):
    def body(pa_ref, pb_ref, b2_ref, w3a_ref, w3b_ref, o_ref):
        ha = jnp.maximum(pa_ref[0] + pa_ref[1] + b2_ref[0, :128], 0.0)
        hb = jnp.maximum(pb_ref[0] + pb_ref[1] + b2_ref[0, 128:], 0.0)
        o_ref[...] = (jnp.dot(ha, w3a_ref[...], preferred_element_type=jnp.float32)
                      + jnp.dot(hb, w3b_ref[...], preferred_element_type=jnp.float32))

    return pl.pallas_call(
        body,
        grid=(_N // _RB,),
        in_specs=[_part_spec(128), _part_spec(128), _full_spec((1, 256)),
                  _full_spec((128, 128)), _full_spec((128, 128))],
        out_specs=pl.BlockSpec((_RB, 128), lambda i: (i, 0)),
        out_shape=jax.ShapeDtypeStruct((_N, 128), jnp.float32),
    )(pa, pb, b2, w3a, w3b)


def _tc_matmul_stage(p, b, w):
    din, dout = w.shape

    def body(p_ref, b_ref, w_ref, o_ref):
        h = jnp.maximum(p_ref[0] + p_ref[1] + b_ref[...], 0.0)
        o_ref[...] = jnp.dot(h, w_ref[...], preferred_element_type=jnp.float32)

    return pl.pallas_call(
        body,
        grid=(_N // _RB,),
        in_specs=[_part_spec(din), _full_spec((1, din)), _full_spec((din, dout))],
        out_specs=pl.BlockSpec((_RB, dout), lambda i: (i, 0)),
        out_shape=jax.ShapeDtypeStruct((_N, dout), jnp.float32),
    )(p, b, w)


def _tc_final(p, b):
    d = p.shape[-1]

    def body(p_ref, b_ref, o_ref):
        o_ref[...] = p_ref[0] + p_ref[1] + b_ref[...]

    return pl.pallas_call(
        body,
        grid=(_N // _RB,),
        in_specs=[_part_spec(d), _full_spec((1, d))],
        out_specs=pl.BlockSpec((_RB, d), lambda i: (i, 0)),
        out_shape=jax.ShapeDtypeStruct((_N, d), jnp.float32),
    )(p, b)


def kernel(x, edge_index, edge_weight, W1, b1, W2, b2, W3, b3, W4, b4, W5, b5):
    src = edge_index[0]
    dst = edge_index[1]

    px = _sc_spmm(x, src, dst, edge_weight)
    s2a, s2b = _tc_stage1(px, W1, b1.reshape(1, -1), W2)

    pa = _sc_spmm(s2a, src, dst, edge_weight)
    pb = _sc_spmm(s2b, src, dst, edge_weight)
    s3 = _tc_stage2(pa, pb, b2.reshape(1, -1), W3[:128], W3[128:])

    p3 = _sc_spmm(s3, src, dst, edge_weight)
    s4 = _tc_matmul_stage(p3, b3.reshape(1, -1), W4)

    p4 = _sc_spmm(s4, src, dst, edge_weight)
    s5 = _tc_matmul_stage(p4, b4.reshape(1, -1), W5)

    p5 = _sc_spmm(s5, src, dst, edge_weight)
    return _tc_final(p5, b5.reshape(1, -1))
